# baseline (speedup 1.0000x reference)
import functools

import jax
import jax.numpy as jnp
import numpy as np
from jax import lax
from jax.experimental import pallas as pl
from jax.experimental.pallas import tpu as pltpu
from jax.experimental.pallas import tpu_sc as plsc

VOCAB = 100000
EMBED = 64
BATCH = 1024
CTX = 20

NC = 2
NS = 16
L = 16
NW = NC * NS


def _sc_avg_t(idx_ctx_major, emb_t):
    mesh = plsc.VectorSubcoreMesh(core_axis_name="c", subcore_axis_name="s")

    @functools.partial(
        pl.kernel,
        mesh=mesh,
        out_type=jax.ShapeDtypeStruct((EMBED, BATCH), jnp.float32),
        compiler_params=pltpu.CompilerParams(
            use_tc_tiling_on_sc=True, needs_layout_passes=False
        ),
        scratch_types=[
            pltpu.VMEM((BATCH * CTX,), jnp.int32),
            pltpu.VMEM((VOCAB,), jnp.float32),
            pltpu.VMEM((BATCH,), jnp.float32),
            pltpu.SemaphoreType.DMA,
        ],
    )
    def k(emb_hbm, idx_hbm, out_hbm, idx_v, row_v, acc_v, sem):
        wid = lax.axis_index("s") * NC + lax.axis_index("c")
        pltpu.sync_copy(idx_hbm, idx_v)
        inv = jnp.float32(1.0 / CTX)
        for dpass in range(EMBED // NW):
            d = wid + dpass * NW
            pltpu.async_copy(emb_hbm.at[d], row_v, sem).wait()

            @pl.loop(0, BATCH // L)
            def _(blk):
                b0 = blk * L
                acc = jnp.zeros((L,), jnp.float32)
                for j in range(CTX):
                    idx16 = idx_v[pl.ds(j * BATCH + b0, L)]
                    acc = acc + plsc.load_gather(row_v, [idx16])
                acc_v[pl.ds(b0, L)] = acc * inv

            pltpu.sync_copy(acc_v, out_hbm.at[d])

    return k(emb_t, idx_ctx_major)


VB = 4096


def _tc_project_t(avg_t, W_t, b2):
    v_local = W_t.shape[1]
    nv = (v_local + VB - 1) // VB

    def body(avg_ref, wt_ref, b_ref, o_ref):
        bcol = b_ref[...].reshape(VB, 1)
        o_ref[...] = (
            lax.dot_general(
                wt_ref[...].astype(jnp.bfloat16),
                avg_ref[...].astype(jnp.bfloat16),
                dimension_numbers=(((0,), (0,)), ((), ())),
                preferred_element_type=jnp.float32,
            )
            + bcol
        )

    return pl.pallas_call(
        body,
        grid=(nv,),
        in_specs=[
            pl.BlockSpec((EMBED, BATCH), lambda i: (0, 0)),
            pl.BlockSpec((EMBED, VB), lambda i: (0, i)),
            pl.BlockSpec((1, VB), lambda i: (0, i)),
        ],
        out_specs=pl.BlockSpec((VB, BATCH), lambda i: (i, 0)),
        out_shape=jax.ShapeDtypeStruct((v_local, BATCH), jnp.float32),
    )(avg_t, W_t, b2)


def _one_core(idx, emb_t, w_t, b2):
    avg_t = _sc_avg_t(idx, emb_t)
    return _tc_project_t(avg_t, w_t, b2)


def kernel(context_words, emb_table, W, b):
    idx = context_words.T.reshape(-1).astype(jnp.int32)
    emb_t = emb_table.T
    w_t = W.T
    b2 = b.reshape(1, VOCAB)
    devs = jax.devices()
    if len(devs) >= 2:
        mesh = jax.sharding.Mesh(np.asarray(devs[:2]), ("v",))
        f = jax.shard_map(
            _one_core,
            mesh=mesh,
            in_specs=(
                jax.sharding.PartitionSpec(),
                jax.sharding.PartitionSpec(),
                jax.sharding.PartitionSpec(None, "v"),
                jax.sharding.PartitionSpec(None, "v"),
            ),
            out_specs=jax.sharding.PartitionSpec("v", None),
            check_vma=False,
        )
        out_t = f(idx, emb_t, w_t, b2)
    else:
        out_t = _one_core(idx, emb_t, w_t, b2)
    return out_t.T


# speedup vs baseline: 2.6300x; 2.6300x over previous
import functools

import jax
import jax.numpy as jnp
import numpy as np
from jax import lax
from jax.experimental import pallas as pl
from jax.experimental.pallas import tpu as pltpu
from jax.experimental.pallas import tpu_sc as plsc

VOCAB = 100000
EMBED = 64
BATCH = 1024
CTX = 20

NC = 2
NS = 16
L = 16
NW = NC * NS


def _sc_avg_t(idx_t, emb_t):
    mesh = plsc.VectorSubcoreMesh(core_axis_name="c", subcore_axis_name="s")

    @functools.partial(
        pl.kernel,
        mesh=mesh,
        out_type=jax.ShapeDtypeStruct((EMBED, BATCH), jnp.float32),
        compiler_params=pltpu.CompilerParams(
            use_tc_tiling_on_sc=True, needs_layout_passes=False
        ),
        scratch_types=[
            pltpu.VMEM((CTX, BATCH), jnp.int32),
            pltpu.VMEM((VOCAB,), jnp.float32),
            pltpu.VMEM((BATCH,), jnp.float32),
            pltpu.SemaphoreType.DMA,
            pltpu.SemaphoreType.DMA,
        ],
    )
    def k(emb_hbm, idx_hbm, out_hbm, idx_v, row_v, acc_v, sem, isem):
        wid = lax.axis_index("s") * NC + lax.axis_index("c")
        icopy = pltpu.async_copy(idx_hbm, idx_v, isem)
        row = pltpu.async_copy(emb_hbm.at[wid], row_v, sem)
        icopy.wait()
        inv = jnp.float32(1.0 / CTX)
        for dpass in range(EMBED // NW):
            d = wid + dpass * NW
            row.wait()

            @pl.loop(0, BATCH // L)
            def _(blk):
                b0 = blk * L
                acc = jnp.zeros((L,), jnp.float32)
                for j in range(CTX):
                    idx16 = idx_v[j, pl.ds(b0, L)]
                    acc = acc + plsc.load_gather(row_v, [idx16])
                acc_v[pl.ds(b0, L)] = acc * inv

            pltpu.sync_copy(acc_v, out_hbm.at[d])
            if dpass + 1 < EMBED // NW:
                row = pltpu.async_copy(emb_hbm.at[d + NW], row_v, sem)

    return k(emb_t, idx_t)


VB = 4096


def _tc_project_t(avg_t, W_t, b2):
    v_local = W_t.shape[1]
    nv = (v_local + VB - 1) // VB

    def body(avg_ref, wt_ref, b_ref, o_ref):
        bcol = b_ref[...].reshape(VB, 1)
        o_ref[...] = (
            lax.dot_general(
                wt_ref[...].astype(jnp.bfloat16),
                avg_ref[...].astype(jnp.bfloat16),
                dimension_numbers=(((0,), (0,)), ((), ())),
                preferred_element_type=jnp.float32,
            )
            + bcol
        )

    return pl.pallas_call(
        body,
        grid=(nv,),
        in_specs=[
            pl.BlockSpec((EMBED, BATCH), lambda i: (0, 0)),
            pl.BlockSpec((EMBED, VB), lambda i: (0, i)),
            pl.BlockSpec((1, VB), lambda i: (0, i)),
        ],
        out_specs=pl.BlockSpec((VB, BATCH), lambda i: (i, 0)),
        out_shape=jax.ShapeDtypeStruct((v_local, BATCH), jnp.float32),
    )(avg_t, W_t, b2)


def _one_core(idx, emb_t, w_t, b2):
    avg_t = _sc_avg_t(idx, emb_t)
    return _tc_project_t(avg_t, w_t, b2)


def kernel(context_words, emb_table, W, b):
    idx_t = context_words.T.astype(jnp.int32)
    out_t = _one_core(idx_t, emb_table.T, W.T, b.reshape(1, VOCAB))
    return out_t.T


# speedup vs baseline: 2.6313x; 1.0005x over previous
import functools

import jax
import jax.numpy as jnp
import numpy as np
from jax import lax
from jax.experimental import pallas as pl
from jax.experimental.pallas import tpu as pltpu
from jax.experimental.pallas import tpu_sc as plsc

VOCAB = 100000
EMBED = 64
BATCH = 1024
CTX = 20

NC = 2
NS = 16
L = 16
NW = NC * NS


def _sc_avg_t(idx_t, emb_t):
    mesh = plsc.VectorSubcoreMesh(core_axis_name="c", subcore_axis_name="s")

    @functools.partial(
        pl.kernel,
        mesh=mesh,
        out_type=jax.ShapeDtypeStruct((EMBED, BATCH), jnp.float32),
        compiler_params=pltpu.CompilerParams(
            use_tc_tiling_on_sc=True, needs_layout_passes=False
        ),
        scratch_types=[
            pltpu.VMEM((CTX, BATCH), jnp.int32),
            pltpu.VMEM((VOCAB,), jnp.float32),
            pltpu.VMEM((BATCH,), jnp.float32),
            pltpu.SemaphoreType.DMA,
            pltpu.SemaphoreType.DMA,
        ],
    )
    def k(emb_hbm, idx_hbm, out_hbm, idx_v, row_v, acc_v, sem, isem):
        wid = lax.axis_index("s") * NC + lax.axis_index("c")
        icopy = pltpu.async_copy(idx_hbm, idx_v, isem)
        row = pltpu.async_copy(emb_hbm.at[wid], row_v, sem)
        icopy.wait()
        inv = jnp.float32(1.0 / CTX)
        for dpass in range(EMBED // NW):
            d = wid + dpass * NW
            row.wait()

            @pl.loop(0, BATCH // L)
            def _(blk):
                b0 = blk * L
                acc = jnp.zeros((L,), jnp.float32)
                for j in range(CTX):
                    idx16 = idx_v[j, pl.ds(b0, L)]
                    acc = acc + plsc.load_gather(row_v, [idx16])
                acc_v[pl.ds(b0, L)] = acc * inv

            pltpu.sync_copy(acc_v, out_hbm.at[d])
            if dpass + 1 < EMBED // NW:
                row = pltpu.async_copy(emb_hbm.at[d + NW], row_v, sem)

    return k(emb_t, idx_t)


VB = 4096


def _tc_project_t(avg_t, W_t, b2):
    v_local = W_t.shape[1]
    nv = (v_local + VB - 1) // VB

    def body(avg_ref, wt_ref, b_ref, o_ref):
        bcol = b_ref[...].reshape(VB, 1)
        o_ref[...] = (
            lax.dot_general(
                wt_ref[...].astype(jnp.bfloat16),
                avg_ref[...].astype(jnp.bfloat16),
                dimension_numbers=(((0,), (0,)), ((), ())),
                preferred_element_type=jnp.float32,
            )
            + bcol
        )

    return pl.pallas_call(
        body,
        grid=(nv,),
        in_specs=[
            pl.BlockSpec((EMBED, BATCH), lambda i: (0, 0)),
            pl.BlockSpec((EMBED, VB), lambda i: (0, i)),
            pl.BlockSpec((VB,), lambda i: (i,)),
        ],
        out_specs=pl.BlockSpec((VB, BATCH), lambda i: (i, 0)),
        out_shape=jax.ShapeDtypeStruct((v_local, BATCH), jnp.float32),
    )(avg_t, W_t, b2)


def _one_core(idx, emb_t, w_t, b2):
    avg_t = _sc_avg_t(idx, emb_t)
    return _tc_project_t(avg_t, w_t, b2)


def kernel(context_words, emb_table, W, b):
    idx_t = context_words.T.astype(jnp.int32)
    out_t = _one_core(idx_t, emb_table.T, W.T, b)
    return out_t.T


# speedup vs baseline: 2.6358x; 1.0017x over previous
import functools

import jax
import jax.numpy as jnp
from jax import lax
from jax.experimental import pallas as pl
from jax.experimental.pallas import tpu as pltpu
from jax.experimental.pallas import tpu_sc as plsc

VOCAB = 100000
EMBED = 64
BATCH = 1024
CTX = 20

NC = 2
NS = 16
L = 16
NW = NC * NS


def _sc_avg_t(idx_t, emb_t):
    mesh = plsc.VectorSubcoreMesh(core_axis_name="c", subcore_axis_name="s")

    @functools.partial(
        pl.kernel,
        mesh=mesh,
        out_type=jax.ShapeDtypeStruct((EMBED, BATCH), jnp.float32),
        compiler_params=pltpu.CompilerParams(
            use_tc_tiling_on_sc=True, needs_layout_passes=False
        ),
        scratch_types=[
            pltpu.VMEM((CTX, BATCH), jnp.int32),
            pltpu.VMEM((VOCAB,), jnp.float32),
            pltpu.VMEM((BATCH,), jnp.float32),
            pltpu.SemaphoreType.DMA,
            pltpu.SemaphoreType.DMA,
        ],
    )
    def k(emb_hbm, idx_hbm, out_hbm, idx_v, row_v, acc_v, sem, isem):
        wid = lax.axis_index("s") * NC + lax.axis_index("c")
        icopy = pltpu.async_copy(idx_hbm, idx_v, isem)
        row = pltpu.async_copy(emb_hbm.at[wid], row_v, sem)
        icopy.wait()
        inv = jnp.float32(1.0 / CTX)
        for dpass in range(EMBED // NW):
            d = wid + dpass * NW
            row.wait()

            @pl.loop(0, BATCH // L)
            def _(blk):
                b0 = blk * L
                acc = jnp.zeros((L,), jnp.float32)
                for j in range(CTX):
                    idx16 = idx_v[j, pl.ds(b0, L)]
                    acc = acc + plsc.load_gather(row_v, [idx16])
                acc_v[pl.ds(b0, L)] = acc * inv

            pltpu.sync_copy(acc_v, out_hbm.at[d])
            if dpass + 1 < EMBED // NW:
                row = pltpu.async_copy(emb_hbm.at[d + NW], row_v, sem)

    return k(emb_t, idx_t)


VB = 4096


def _tc_project_t(avg_t, W_t, b2):
    v_local = W_t.shape[1]
    nv = (v_local + VB - 1) // VB

    def body(avg_ref, wt_ref, b_ref, o_ref):
        bcol = b_ref[...].reshape(VB, 1)
        o_ref[...] = (
            lax.dot_general(
                wt_ref[...].astype(jnp.bfloat16),
                avg_ref[...].astype(jnp.bfloat16),
                dimension_numbers=(((0,), (0,)), ((), ())),
                preferred_element_type=jnp.float32,
            )
            + bcol
        )

    return pl.pallas_call(
        body,
        grid=(nv,),
        in_specs=[
            pl.BlockSpec((EMBED, BATCH), lambda i: (0, 0)),
            pl.BlockSpec((EMBED, VB), lambda i: (0, i)),
            pl.BlockSpec((VB,), lambda i: (i,)),
        ],
        out_specs=pl.BlockSpec((VB, BATCH), lambda i: (i, 0)),
        out_shape=jax.ShapeDtypeStruct((v_local, BATCH), jnp.float32),
    )(avg_t, W_t, b2)


def _one_core(idx, emb_t, w_t, b2):
    avg_t = _sc_avg_t(idx, emb_t)
    return _tc_project_t(avg_t, w_t, b2)


def kernel(context_words, emb_table, W, b):
    idx_t = context_words.T.astype(jnp.int32)
    out_t = _one_core(idx_t, emb_table.T, W.T, b)
    return out_t.T
